# baseline (speedup 1.0000x reference)
import functools

import jax
import jax.numpy as jnp
from jax import lax
from jax.experimental import pallas as pl
from jax.experimental.pallas import tpu as pltpu
from jax.experimental.pallas import tpu_sc as plsc

B, S, H, E, V = 32, 128, 1024, 256, 8192


def _build_sc_gather(n_rows: int):
    info = plsc.get_sparse_core_info()
    nc, ns = info.num_cores, info.num_subcores
    nw = nc * ns
    assert n_rows % (8 * nw) == 0
    rows_per_w = n_rows // nw

    @functools.partial(
        pl.kernel,
        out_type=jax.ShapeDtypeStruct((n_rows, E), jnp.float32),
        mesh=plsc.VectorSubcoreMesh(core_axis_name="c", subcore_axis_name="s"),
        scratch_types=[
            pltpu.VMEM((rows_per_w,), jnp.int32),
            pltpu.VMEM((rows_per_w, E), jnp.float32),
            pltpu.SemaphoreType.DMA,
        ],
    )
    def gather(table_hbm, idx_hbm, out_hbm, idx_v, rows_v, sem):
        wid = lax.axis_index("s") * nc + lax.axis_index("c")
        base = wid * rows_per_w
        pltpu.sync_copy(idx_hbm.at[pl.ds(base, rows_per_w)], idx_v)
        pltpu.async_copy(table_hbm.at[idx_v], rows_v, sem).wait()
        pltpu.sync_copy(rows_v, out_hbm.at[pl.ds(base, rows_per_w)])

    return gather


_DN_T = (((1,), (1,)), ((), ()))


def _gi_body(x_ref, w_ref, b_ref, o_ref):
    o_ref[...] = (
        lax.dot_general(
            x_ref[...], w_ref[...], _DN_T, preferred_element_type=jnp.float32
        )
        + b_ref[...]
    )


def _input_proj(emb, w_ih, b_ih2, interpret=False):
    m = emb.shape[0]
    m_blk = 512
    grid = (m // m_blk,)
    return pl.pallas_call(
        _gi_body,
        grid=grid,
        in_specs=[
            pl.BlockSpec((m_blk, E), lambda i: (i, 0)),
            pl.BlockSpec((3 * H, E), lambda i: (0, 0)),
            pl.BlockSpec((1, 3 * H), lambda i: (0, 0)),
        ],
        out_specs=pl.BlockSpec((m_blk, 3 * H), lambda i: (i, 0)),
        out_shape=jax.ShapeDtypeStruct((m, 3 * H), jnp.float32),
        interpret=interpret,
    )(emb, w_ih, b_ih2)


def _gru_body(gi_ref, w_ref, b_ref, o_ref, h_ref):
    t = pl.program_id(0)

    @pl.when(t == 0)
    def _():
        h_ref[...] = jnp.zeros_like(h_ref)

    h = h_ref[...]
    gh = (
        lax.dot_general(h, w_ref[...], _DN_T, preferred_element_type=jnp.float32)
        + b_ref[...]
    )
    gi = gi_ref[0]
    r = jax.nn.sigmoid(gi[:, :H] + gh[:, :H])
    z = jax.nn.sigmoid(gi[:, H : 2 * H] + gh[:, H : 2 * H])
    n = jnp.tanh(gi[:, 2 * H :] + r * gh[:, 2 * H :])
    h_new = (1.0 - z) * n + z * h
    h_ref[...] = h_new
    o_ref[0] = h_new


def _gru(gi_all, w_hh, b_hh2, interpret=False):
    return pl.pallas_call(
        _gru_body,
        grid=(S,),
        in_specs=[
            pl.BlockSpec((1, B, 3 * H), lambda t: (t, 0, 0)),
            pl.BlockSpec((3 * H, H), lambda t: (0, 0)),
            pl.BlockSpec((1, 3 * H), lambda t: (0, 0)),
        ],
        out_specs=pl.BlockSpec((1, B, H), lambda t: (t, 0, 0)),
        out_shape=jax.ShapeDtypeStruct((S, B, H), jnp.float32),
        scratch_shapes=[pltpu.VMEM((B, H), jnp.float32)],
        compiler_params=pltpu.CompilerParams(
            dimension_semantics=("arbitrary",)
        ),
        interpret=interpret,
    )(gi_all, w_hh, b_hh2)


_S_BLK = 8
_N_BLK = 1024


def _fc_body(x_ref, w_ref, b_ref, o_ref):
    x = x_ref[...]
    xt = jnp.swapaxes(x, 0, 1).reshape(B * _S_BLK, H)
    y = (
        lax.dot_general(xt, w_ref[...], _DN_T, preferred_element_type=jnp.float32)
        + b_ref[...]
    )
    o_ref[...] = y.reshape(B, _S_BLK, _N_BLK)


def _fc(outs, w_fc, b_fc2, interpret=False):
    grid = (V // _N_BLK, S // _S_BLK)
    return pl.pallas_call(
        _fc_body,
        grid=grid,
        in_specs=[
            pl.BlockSpec((_S_BLK, B, H), lambda n, s: (s, 0, 0)),
            pl.BlockSpec((_N_BLK, H), lambda n, s: (n, 0)),
            pl.BlockSpec((1, _N_BLK), lambda n, s: (0, n)),
        ],
        out_specs=pl.BlockSpec((B, _S_BLK, _N_BLK), lambda n, s: (0, s, n)),
        out_shape=jax.ShapeDtypeStruct((B, S, V), jnp.float32),
        compiler_params=pltpu.CompilerParams(
            dimension_semantics=("parallel", "arbitrary")
        ),
        interpret=interpret,
    )(outs, w_fc, b_fc2)


def kernel(embed_table, W_ih, W_hh, b_ih, b_hh, W_fc, b_fc, y_inp):
    idx = jnp.transpose(y_inp).reshape(-1).astype(jnp.int32)

    emb = _build_sc_gather(S * B)(embed_table, idx)

    gi = _input_proj(emb, W_ih, b_ih.reshape(1, 3 * H))
    outs = _gru(
        gi.reshape(S, B, 3 * H), W_hh, b_hh.reshape(1, 3 * H)
    )
    logits = _fc(outs, W_fc, b_fc.reshape(1, V))
    h_last = outs[S - 1][None]
    return (logits, h_last)


# speedup vs baseline: 1.1678x; 1.1678x over previous
import functools

import jax
import jax.numpy as jnp
from jax import lax
from jax.experimental import pallas as pl
from jax.experimental.pallas import tpu as pltpu
from jax.experimental.pallas import tpu_sc as plsc

B, S, H, E, V = 32, 128, 1024, 256, 8192


def _build_sc_gather(n_rows: int):
    info = plsc.get_sparse_core_info()
    nc, ns = info.num_cores, info.num_subcores
    nw = nc * ns
    assert n_rows % (8 * nw) == 0
    rows_per_w = n_rows // nw

    @functools.partial(
        pl.kernel,
        out_type=jax.ShapeDtypeStruct((n_rows, E), jnp.float32),
        mesh=plsc.VectorSubcoreMesh(core_axis_name="c", subcore_axis_name="s"),
        scratch_types=[
            pltpu.VMEM((rows_per_w,), jnp.int32),
            pltpu.VMEM((rows_per_w, E), jnp.float32),
            pltpu.SemaphoreType.DMA,
        ],
    )
    def gather(table_hbm, idx_hbm, out_hbm, idx_v, rows_v, sem):
        wid = lax.axis_index("s") * nc + lax.axis_index("c")
        base = wid * rows_per_w
        pltpu.sync_copy(idx_hbm.at[pl.ds(base, rows_per_w)], idx_v)
        pltpu.async_copy(table_hbm.at[idx_v], rows_v, sem).wait()
        pltpu.sync_copy(rows_v, out_hbm.at[pl.ds(base, rows_per_w)])

    return gather


def _gi_body(x_ref, w_ref, b_ref, o_ref):
    x = x_ref[...].astype(jnp.bfloat16)
    o_ref[...] = (
        jnp.dot(x, w_ref[...], preferred_element_type=jnp.float32) + b_ref[...]
    )


def _input_proj(emb, w_ihT, b_ih2, interpret=False):
    m = emb.shape[0]
    m_blk = 512
    grid = (m // m_blk,)
    return pl.pallas_call(
        _gi_body,
        grid=grid,
        in_specs=[
            pl.BlockSpec((m_blk, E), lambda i: (i, 0)),
            pl.BlockSpec((E, 3 * H), lambda i: (0, 0)),
            pl.BlockSpec((1, 3 * H), lambda i: (0, 0)),
        ],
        out_specs=pl.BlockSpec((m_blk, 3 * H), lambda i: (i, 0)),
        out_shape=jax.ShapeDtypeStruct((m, 3 * H), jnp.float32),
        interpret=interpret,
    )(emb, w_ihT, b_ih2)


def _gru_body(gi_ref, w_ref, b_ref, o_ref, h_ref):
    t = pl.program_id(0)

    @pl.when(t == 0)
    def _():
        h_ref[...] = jnp.zeros_like(h_ref)

    h = h_ref[...]
    gh = (
        jnp.dot(
            h.astype(jnp.bfloat16), w_ref[...], preferred_element_type=jnp.float32
        )
        + b_ref[...]
    )
    gi = gi_ref[0]
    r = jax.nn.sigmoid(gi[:, :H] + gh[:, :H])
    z = jax.nn.sigmoid(gi[:, H : 2 * H] + gh[:, H : 2 * H])
    n = jnp.tanh(gi[:, 2 * H :] + r * gh[:, 2 * H :])
    h_new = (1.0 - z) * n + z * h
    h_ref[...] = h_new
    o_ref[0] = h_new


def _gru(gi_all, w_hhT, b_hh2, interpret=False):
    return pl.pallas_call(
        _gru_body,
        grid=(S,),
        in_specs=[
            pl.BlockSpec((1, B, 3 * H), lambda t: (t, 0, 0)),
            pl.BlockSpec((H, 3 * H), lambda t: (0, 0)),
            pl.BlockSpec((1, 3 * H), lambda t: (0, 0)),
        ],
        out_specs=pl.BlockSpec((1, B, H), lambda t: (t, 0, 0)),
        out_shape=jax.ShapeDtypeStruct((S, B, H), jnp.float32),
        scratch_shapes=[pltpu.VMEM((B, H), jnp.float32)],
        compiler_params=pltpu.CompilerParams(
            dimension_semantics=("arbitrary",)
        ),
        interpret=interpret,
    )(gi_all, w_hhT, b_hh2)


_S_BLK = 8
_N_BLK = 1024


def _fc_body(x_ref, w_ref, b_ref, o_ref):
    x = x_ref[...]
    xt = jnp.swapaxes(x, 0, 1).reshape(B * _S_BLK, H).astype(jnp.bfloat16)
    y = jnp.dot(xt, w_ref[...], preferred_element_type=jnp.float32) + b_ref[...]
    o_ref[...] = y.reshape(B, _S_BLK, _N_BLK)


def _fc(outs, w_fcT, b_fc2, interpret=False):
    grid = (V // _N_BLK, S // _S_BLK)
    return pl.pallas_call(
        _fc_body,
        grid=grid,
        in_specs=[
            pl.BlockSpec((_S_BLK, B, H), lambda n, s: (s, 0, 0)),
            pl.BlockSpec((H, _N_BLK), lambda n, s: (0, n)),
            pl.BlockSpec((1, _N_BLK), lambda n, s: (0, n)),
        ],
        out_specs=pl.BlockSpec((B, _S_BLK, _N_BLK), lambda n, s: (0, s, n)),
        out_shape=jax.ShapeDtypeStruct((B, S, V), jnp.float32),
        compiler_params=pltpu.CompilerParams(
            dimension_semantics=("parallel", "arbitrary")
        ),
        interpret=interpret,
    )(outs, w_fcT, b_fc2)


def kernel(embed_table, W_ih, W_hh, b_ih, b_hh, W_fc, b_fc, y_inp):
    idx = jnp.transpose(y_inp).reshape(-1).astype(jnp.int32)

    emb = _build_sc_gather(S * B)(embed_table, idx)

    bf = jnp.bfloat16
    gi = _input_proj(
        emb, jnp.transpose(W_ih).astype(bf), b_ih.reshape(1, 3 * H)
    )
    outs = _gru(
        gi.reshape(S, B, 3 * H),
        jnp.transpose(W_hh).astype(bf),
        b_hh.reshape(1, 3 * H),
    )
    logits = _fc(
        outs, jnp.transpose(W_fc).astype(bf), b_fc.reshape(1, V)
    )
    h_last = outs[S - 1][None]
    return (logits, h_last)


# speedup vs baseline: 1.2176x; 1.0426x over previous
import functools

import jax
import jax.numpy as jnp
from jax import lax
from jax.experimental import pallas as pl
from jax.experimental.pallas import tpu as pltpu
from jax.experimental.pallas import tpu_sc as plsc

B, S, H, E, V = 32, 128, 1024, 256, 8192


def _build_sc_gather(n_rows: int):
    info = plsc.get_sparse_core_info()
    nc, ns = info.num_cores, info.num_subcores
    nw = nc * ns
    assert n_rows % (8 * nw) == 0
    rows_per_w = n_rows // nw

    @functools.partial(
        pl.kernel,
        out_type=jax.ShapeDtypeStruct((n_rows, E), jnp.float32),
        mesh=plsc.VectorSubcoreMesh(core_axis_name="c", subcore_axis_name="s"),
        scratch_types=[
            pltpu.VMEM((rows_per_w,), jnp.int32),
            pltpu.VMEM((rows_per_w, E), jnp.float32),
            pltpu.SemaphoreType.DMA,
        ],
    )
    def gather(table_hbm, idx_hbm, out_hbm, idx_v, rows_v, sem):
        wid = lax.axis_index("s") * nc + lax.axis_index("c")
        base = wid * rows_per_w
        pltpu.sync_copy(idx_hbm.at[pl.ds(base, rows_per_w)], idx_v)
        pltpu.async_copy(table_hbm.at[idx_v], rows_v, sem).wait()
        pltpu.sync_copy(rows_v, out_hbm.at[pl.ds(base, rows_per_w)])

    return gather


def _gi_body(x_ref, w_ref, b_ref, o_ref):
    x = x_ref[...].astype(jnp.bfloat16)
    o_ref[...] = (
        jnp.dot(x, w_ref[...], preferred_element_type=jnp.float32) + b_ref[...]
    )


def _input_proj(emb, w_ihT, b_ih2, interpret=False):
    m = emb.shape[0]
    m_blk = 512
    grid = (m // m_blk,)
    return pl.pallas_call(
        _gi_body,
        grid=grid,
        in_specs=[
            pl.BlockSpec((m_blk, E), lambda i: (i, 0)),
            pl.BlockSpec((E, 3 * H), lambda i: (0, 0)),
            pl.BlockSpec((1, 3 * H), lambda i: (0, 0)),
        ],
        out_specs=pl.BlockSpec((m_blk, 3 * H), lambda i: (i, 0)),
        out_shape=jax.ShapeDtypeStruct((m, 3 * H), jnp.float32),
        interpret=interpret,
    )(emb, w_ihT, b_ih2)


def _gru_body(gi_ref, w_ref, b_ref, o_ref, h_ref):
    t = pl.program_id(0)

    @pl.when(t == 0)
    def _():
        h_ref[...] = jnp.zeros_like(h_ref)

    h = h_ref[...]
    gh = (
        jnp.dot(
            h.astype(jnp.bfloat16), w_ref[...], preferred_element_type=jnp.float32
        )
        + b_ref[...]
    )
    gi = gi_ref[0]
    r = jax.nn.sigmoid(gi[:, :H] + gh[:, :H])
    z = jax.nn.sigmoid(gi[:, H : 2 * H] + gh[:, H : 2 * H])
    n = jnp.tanh(gi[:, 2 * H :] + r * gh[:, 2 * H :])
    h_new = (1.0 - z) * n + z * h
    h_ref[...] = h_new
    o_ref[0] = h_new


def _gru(gi_all, w_hhT, b_hh2, interpret=False):
    return pl.pallas_call(
        _gru_body,
        grid=(S,),
        in_specs=[
            pl.BlockSpec((1, B, 3 * H), lambda t: (t, 0, 0)),
            pl.BlockSpec((H, 3 * H), lambda t: (0, 0)),
            pl.BlockSpec((1, 3 * H), lambda t: (0, 0)),
        ],
        out_specs=pl.BlockSpec((1, B, H), lambda t: (t, 0, 0)),
        out_shape=jax.ShapeDtypeStruct((S, B, H), jnp.float32),
        scratch_shapes=[pltpu.VMEM((B, H), jnp.float32)],
        compiler_params=pltpu.CompilerParams(
            dimension_semantics=("arbitrary",)
        ),
        interpret=interpret,
    )(gi_all, w_hhT, b_hh2)


_S_BLK = 8
_N_BLK = 1024


def _fc_body(x_ref, w_ref, b_ref, o_ref):
    x = x_ref[...]
    xt = jnp.swapaxes(x, 0, 1).reshape(B * _S_BLK, H)
    y = (
        lax.dot_general(
            xt, w_ref[...], (((1,), (1,)), ((), ())),
            preferred_element_type=jnp.float32,
        )
        + b_ref[...]
    )
    o_ref[...] = y.reshape(B, _S_BLK, _N_BLK)


def _fc(outs, w_fc, b_fc2, interpret=False):
    grid = (V // _N_BLK, S // _S_BLK)
    return pl.pallas_call(
        _fc_body,
        grid=grid,
        in_specs=[
            pl.BlockSpec((_S_BLK, B, H), lambda n, s: (s, 0, 0)),
            pl.BlockSpec((_N_BLK, H), lambda n, s: (n, 0)),
            pl.BlockSpec((1, _N_BLK), lambda n, s: (0, n)),
        ],
        out_specs=pl.BlockSpec((B, _S_BLK, _N_BLK), lambda n, s: (0, s, n)),
        out_shape=jax.ShapeDtypeStruct((B, S, V), jnp.float32),
        compiler_params=pltpu.CompilerParams(
            dimension_semantics=("parallel", "arbitrary")
        ),
        interpret=interpret,
    )(outs, w_fc, b_fc2)


def kernel(embed_table, W_ih, W_hh, b_ih, b_hh, W_fc, b_fc, y_inp):
    idx = jnp.transpose(y_inp).reshape(-1).astype(jnp.int32)

    emb = _build_sc_gather(S * B)(embed_table, idx)

    bf = jnp.bfloat16
    gi = _input_proj(
        emb, jnp.transpose(W_ih).astype(bf), b_ih.reshape(1, 3 * H)
    )
    outs = _gru(
        gi.reshape(S, B, 3 * H),
        jnp.transpose(W_hh).astype(bf),
        b_hh.reshape(1, 3 * H),
    )
    logits = _fc(outs, W_fc, b_fc.reshape(1, V))
    h_last = outs[S - 1][None]
    return (logits, h_last)


# speedup vs baseline: 1.3963x; 1.1468x over previous
import functools

import jax
import jax.numpy as jnp
from jax import lax
from jax.experimental import pallas as pl
from jax.experimental.pallas import tpu as pltpu
from jax.experimental.pallas import tpu_sc as plsc

B, S, H, E, V = 32, 128, 1024, 256, 8192


def _build_sc_gather(n_rows: int):
    info = plsc.get_sparse_core_info()
    nc, ns = info.num_cores, info.num_subcores
    nw = nc * ns
    assert n_rows % (8 * nw) == 0
    rows_per_w = n_rows // nw

    @functools.partial(
        pl.kernel,
        out_type=jax.ShapeDtypeStruct((n_rows, E), jnp.float32),
        mesh=plsc.VectorSubcoreMesh(core_axis_name="c", subcore_axis_name="s"),
        scratch_types=[
            pltpu.VMEM((rows_per_w,), jnp.int32),
            pltpu.VMEM((rows_per_w, E), jnp.float32),
            pltpu.SemaphoreType.DMA,
        ],
    )
    def gather(table_hbm, idx_hbm, out_hbm, idx_v, rows_v, sem):
        wid = lax.axis_index("s") * nc + lax.axis_index("c")
        base = wid * rows_per_w
        pltpu.sync_copy(idx_hbm.at[pl.ds(base, rows_per_w)], idx_v)
        pltpu.async_copy(table_hbm.at[idx_v], rows_v, sem).wait()
        pltpu.sync_copy(rows_v, out_hbm.at[pl.ds(base, rows_per_w)])

    return gather


def _gi_body(x_ref, w_ref, b_ref, o_ref):
    x = x_ref[...].astype(jnp.bfloat16)
    o_ref[...] = (
        jnp.dot(x, w_ref[...], preferred_element_type=jnp.float32) + b_ref[...]
    )


def _input_proj(emb, w_ihT, b_ih2, interpret=False):
    m = emb.shape[0]
    m_blk = 512
    grid = (m // m_blk,)
    return pl.pallas_call(
        _gi_body,
        grid=grid,
        in_specs=[
            pl.BlockSpec((m_blk, E), lambda i: (i, 0)),
            pl.BlockSpec((E, 3 * H), lambda i: (0, 0)),
            pl.BlockSpec((1, 3 * H), lambda i: (0, 0)),
        ],
        out_specs=pl.BlockSpec((m_blk, 3 * H), lambda i: (i, 0)),
        out_shape=jax.ShapeDtypeStruct((m, 3 * H), jnp.float32),
        interpret=interpret,
    )(emb, w_ihT, b_ih2)


_T_BLK = 2


def _gru_body(gi_ref, w_ref, b_ref, o_ref, h_ref):
    t = pl.program_id(0)

    @pl.when(t == 0)
    def _():
        h_ref[...] = jnp.zeros_like(h_ref)

    h = h_ref[...]
    w = w_ref[...]
    b = b_ref[...]
    for k in range(_T_BLK):
        gh = (
            jnp.dot(
                h.astype(jnp.bfloat16), w, preferred_element_type=jnp.float32
            )
            + b
        )
        gi = gi_ref[k]
        r = jax.nn.sigmoid(gi[:, :H] + gh[:, :H])
        z = jax.nn.sigmoid(gi[:, H : 2 * H] + gh[:, H : 2 * H])
        n = jnp.tanh(gi[:, 2 * H :] + r * gh[:, 2 * H :])
        h = (1.0 - z) * n + z * h
        o_ref[k] = h
    h_ref[...] = h


def _gru(gi_all, w_hhT, b_hh2, interpret=False):
    return pl.pallas_call(
        _gru_body,
        grid=(S // _T_BLK,),
        in_specs=[
            pl.BlockSpec((_T_BLK, B, 3 * H), lambda t: (t, 0, 0)),
            pl.BlockSpec((H, 3 * H), lambda t: (0, 0)),
            pl.BlockSpec((1, 3 * H), lambda t: (0, 0)),
        ],
        out_specs=pl.BlockSpec((_T_BLK, B, H), lambda t: (t, 0, 0)),
        out_shape=jax.ShapeDtypeStruct((S, B, H), jnp.float32),
        scratch_shapes=[pltpu.VMEM((B, H), jnp.float32)],
        compiler_params=pltpu.CompilerParams(
            dimension_semantics=("arbitrary",)
        ),
        interpret=interpret,
    )(gi_all, w_hhT, b_hh2)


_S_BLK = 8
_N_BLK = 2048


def _fc_body(x_ref, w_ref, b_ref, o_ref):
    x = x_ref[...]
    xt = jnp.swapaxes(x, 0, 1).reshape(B * _S_BLK, H)
    y = (
        lax.dot_general(
            xt, w_ref[...], (((1,), (1,)), ((), ())),
            preferred_element_type=jnp.float32,
        )
        + b_ref[...]
    )
    o_ref[...] = y.reshape(B, _S_BLK, _N_BLK)


def _fc(outs, w_fc, b_fc2, interpret=False):
    grid = (V // _N_BLK, S // _S_BLK)
    return pl.pallas_call(
        _fc_body,
        grid=grid,
        in_specs=[
            pl.BlockSpec((_S_BLK, B, H), lambda n, s: (s, 0, 0)),
            pl.BlockSpec((_N_BLK, H), lambda n, s: (n, 0)),
            pl.BlockSpec((1, _N_BLK), lambda n, s: (0, n)),
        ],
        out_specs=pl.BlockSpec((B, _S_BLK, _N_BLK), lambda n, s: (0, s, n)),
        out_shape=jax.ShapeDtypeStruct((B, S, V), jnp.float32),
        compiler_params=pltpu.CompilerParams(
            dimension_semantics=("parallel", "arbitrary")
        ),
        interpret=interpret,
    )(outs, w_fc, b_fc2)


def kernel(embed_table, W_ih, W_hh, b_ih, b_hh, W_fc, b_fc, y_inp):
    idx = jnp.transpose(y_inp).reshape(-1).astype(jnp.int32)

    emb = _build_sc_gather(S * B)(embed_table, idx)

    bf = jnp.bfloat16
    gi = _input_proj(
        emb, jnp.transpose(W_ih).astype(bf), b_ih.reshape(1, 3 * H)
    )
    outs = _gru(
        gi.reshape(S, B, 3 * H),
        jnp.transpose(W_hh).astype(bf),
        b_hh.reshape(1, 3 * H),
    )
    logits = _fc(outs, W_fc.astype(bf), b_fc.reshape(1, V))
    h_last = outs[S - 1][None]
    return (logits, h_last)


# speedup vs baseline: 1.5586x; 1.1162x over previous
import functools

import jax
import jax.numpy as jnp
from jax import lax
from jax.experimental import pallas as pl
from jax.experimental.pallas import tpu as pltpu
from jax.experimental.pallas import tpu_sc as plsc

B, S, H, E, V = 32, 128, 1024, 256, 8192


def _build_sc_gather(n_rows: int):
    info = plsc.get_sparse_core_info()
    nc, ns = info.num_cores, info.num_subcores
    nw = nc * ns
    assert n_rows % (8 * nw) == 0
    rows_per_w = n_rows // nw

    @functools.partial(
        pl.kernel,
        out_type=jax.ShapeDtypeStruct((n_rows, E), jnp.float32),
        mesh=plsc.VectorSubcoreMesh(core_axis_name="c", subcore_axis_name="s"),
        scratch_types=[
            pltpu.VMEM((rows_per_w,), jnp.int32),
            pltpu.VMEM((rows_per_w, E), jnp.float32),
            pltpu.SemaphoreType.DMA,
        ],
    )
    def gather(table_hbm, idx_hbm, out_hbm, idx_v, rows_v, sem):
        wid = lax.axis_index("s") * nc + lax.axis_index("c")
        base = wid * rows_per_w
        pltpu.sync_copy(idx_hbm.at[pl.ds(base, rows_per_w)], idx_v)
        pltpu.async_copy(table_hbm.at[idx_v], rows_v, sem).wait()
        pltpu.sync_copy(rows_v, out_hbm.at[pl.ds(base, rows_per_w)])

    return gather


def _gi_body(x_ref, w_ref, b_ref, o_ref):
    x = x_ref[...].astype(jnp.bfloat16)
    o_ref[...] = (
        jnp.dot(x, w_ref[...], preferred_element_type=jnp.float32) + b_ref[...]
    ).astype(jnp.bfloat16)


def _input_proj(emb, w_ihT, b_ih2, interpret=False):
    m = emb.shape[0]
    m_blk = 512
    grid = (m // m_blk,)
    return pl.pallas_call(
        _gi_body,
        grid=grid,
        in_specs=[
            pl.BlockSpec((m_blk, E), lambda i: (i, 0)),
            pl.BlockSpec((E, 3 * H), lambda i: (0, 0)),
            pl.BlockSpec((1, 3 * H), lambda i: (0, 0)),
        ],
        out_specs=pl.BlockSpec((m_blk, 3 * H), lambda i: (i, 0)),
        out_shape=jax.ShapeDtypeStruct((m, 3 * H), jnp.bfloat16),
        interpret=interpret,
    )(emb, w_ihT, b_ih2)


_T_BLK = 4


def _gru_body(gi_ref, w_ref, b_ref, o_ref, h_ref):
    t = pl.program_id(0)

    @pl.when(t == 0)
    def _():
        h_ref[...] = jnp.zeros_like(h_ref)

    h = h_ref[...]
    w = w_ref[...]
    b = b_ref[...]
    for k in range(_T_BLK):
        gh = (
            jnp.dot(
                h.astype(jnp.bfloat16), w, preferred_element_type=jnp.float32
            )
            + b
        )
        gi = gi_ref[k]
        r = jax.nn.sigmoid(gi[:, :H] + gh[:, :H])
        z = jax.nn.sigmoid(gi[:, H : 2 * H] + gh[:, H : 2 * H])
        n = jnp.tanh(gi[:, 2 * H :] + r * gh[:, 2 * H :])
        h = (1.0 - z) * n + z * h
        o_ref[k] = h.astype(jnp.bfloat16)
    h_ref[...] = h


def _gru(gi_all, w_hhT, b_hh2, interpret=False):
    return pl.pallas_call(
        _gru_body,
        grid=(S // _T_BLK,),
        in_specs=[
            pl.BlockSpec((_T_BLK, B, 3 * H), lambda t: (t, 0, 0)),
            pl.BlockSpec((H, 3 * H), lambda t: (0, 0)),
            pl.BlockSpec((1, 3 * H), lambda t: (0, 0)),
        ],
        out_specs=pl.BlockSpec((_T_BLK, B, H), lambda t: (t, 0, 0)),
        out_shape=jax.ShapeDtypeStruct((S, B, H), jnp.bfloat16),
        scratch_shapes=[pltpu.VMEM((B, H), jnp.float32)],
        compiler_params=pltpu.CompilerParams(
            dimension_semantics=("arbitrary",)
        ),
        interpret=interpret,
    )(gi_all, w_hhT, b_hh2)


_S_BLK = 8
_N_BLK = 4096


def _fc_body(x_ref, w_ref, b_ref, o_ref):
    x = x_ref[...]
    xt = jnp.swapaxes(x, 0, 1).reshape(B * _S_BLK, H)
    y = (
        lax.dot_general(
            xt, w_ref[...], (((1,), (1,)), ((), ())),
            preferred_element_type=jnp.float32,
        )
        + b_ref[...]
    )
    o_ref[...] = y.reshape(B, _S_BLK, _N_BLK)


def _fc(outs, w_fc, b_fc2, interpret=False):
    grid = (V // _N_BLK, S // _S_BLK)
    return pl.pallas_call(
        _fc_body,
        grid=grid,
        in_specs=[
            pl.BlockSpec((_S_BLK, B, H), lambda n, s: (s, 0, 0)),
            pl.BlockSpec((_N_BLK, H), lambda n, s: (n, 0)),
            pl.BlockSpec((1, _N_BLK), lambda n, s: (0, n)),
        ],
        out_specs=pl.BlockSpec((B, _S_BLK, _N_BLK), lambda n, s: (0, s, n)),
        out_shape=jax.ShapeDtypeStruct((B, S, V), jnp.float32),
        compiler_params=pltpu.CompilerParams(
            dimension_semantics=("parallel", "arbitrary")
        ),
        interpret=interpret,
    )(outs, w_fc, b_fc2)


def kernel(embed_table, W_ih, W_hh, b_ih, b_hh, W_fc, b_fc, y_inp):
    idx = jnp.transpose(y_inp).reshape(-1).astype(jnp.int32)

    emb = _build_sc_gather(S * B)(embed_table, idx)

    bf = jnp.bfloat16
    gi = _input_proj(
        emb, jnp.transpose(W_ih).astype(bf), b_ih.reshape(1, 3 * H)
    )
    outs = _gru(
        gi.reshape(S, B, 3 * H),
        jnp.transpose(W_hh).astype(bf),
        b_hh.reshape(1, 3 * H),
    )
    logits = _fc(outs, W_fc.astype(bf), b_fc.reshape(1, V))
    h_last = outs[S - 1][None].astype(jnp.float32)
    return (logits, h_last)


# speedup vs baseline: 1.5707x; 1.0078x over previous
import functools

import jax
import jax.numpy as jnp
from jax import lax
from jax.experimental import pallas as pl
from jax.experimental.pallas import tpu as pltpu
from jax.experimental.pallas import tpu_sc as plsc

B, S, H, E, V = 32, 128, 1024, 256, 8192


def _build_sc_gather(n_rows: int):
    info = plsc.get_sparse_core_info()
    nc, ns = info.num_cores, info.num_subcores
    nw = nc * ns
    assert n_rows % (8 * nw) == 0
    rows_per_w = n_rows // nw

    @functools.partial(
        pl.kernel,
        out_type=jax.ShapeDtypeStruct((n_rows, E), jnp.float32),
        mesh=plsc.VectorSubcoreMesh(core_axis_name="c", subcore_axis_name="s"),
        scratch_types=[
            pltpu.VMEM((rows_per_w,), jnp.int32),
            pltpu.VMEM((rows_per_w, E), jnp.float32),
            pltpu.SemaphoreType.DMA,
        ],
    )
    def gather(table_hbm, idx_hbm, out_hbm, idx_v, rows_v, sem):
        wid = lax.axis_index("s") * nc + lax.axis_index("c")
        base = wid * rows_per_w
        pltpu.sync_copy(idx_hbm.at[pl.ds(base, rows_per_w)], idx_v)
        pltpu.async_copy(table_hbm.at[idx_v], rows_v, sem).wait()
        pltpu.sync_copy(rows_v, out_hbm.at[pl.ds(base, rows_per_w)])

    return gather


def _gi_body(x_ref, w_ref, b_ref, o_ref):
    x = x_ref[...].astype(jnp.bfloat16)
    o_ref[...] = (
        jnp.dot(x, w_ref[...], preferred_element_type=jnp.float32) + b_ref[...]
    ).astype(jnp.bfloat16)


def _input_proj(emb, w_ihT, b_ih2, interpret=False):
    m = emb.shape[0]
    m_blk = 512
    grid = (m // m_blk,)
    return pl.pallas_call(
        _gi_body,
        grid=grid,
        in_specs=[
            pl.BlockSpec((m_blk, E), lambda i: (i, 0)),
            pl.BlockSpec((E, 3 * H), lambda i: (0, 0)),
            pl.BlockSpec((1, 3 * H), lambda i: (0, 0)),
        ],
        out_specs=pl.BlockSpec((m_blk, 3 * H), lambda i: (i, 0)),
        out_shape=jax.ShapeDtypeStruct((m, 3 * H), jnp.bfloat16),
        interpret=interpret,
    )(emb, w_ihT, b_ih2)


_T_BLK = 8


def _gru_body(gi_ref, w_ref, b_ref, o_ref, h_ref):
    t = pl.program_id(0)

    @pl.when(t == 0)
    def _():
        h_ref[...] = jnp.zeros_like(h_ref)

    h = h_ref[...]
    w = w_ref[...]
    b = b_ref[...]
    for k in range(_T_BLK):
        gh = (
            jnp.dot(
                h.astype(jnp.bfloat16), w, preferred_element_type=jnp.float32
            )
            + b
        )
        gi = gi_ref[k]
        r = jax.nn.sigmoid(gi[:, :H] + gh[:, :H])
        z = jax.nn.sigmoid(gi[:, H : 2 * H] + gh[:, H : 2 * H])
        n = jnp.tanh(gi[:, 2 * H :] + r * gh[:, 2 * H :])
        h = (1.0 - z) * n + z * h
        o_ref[k] = h.astype(jnp.bfloat16)
    h_ref[...] = h


def _gru(gi_all, w_hhT, b_hh2, interpret=False):
    return pl.pallas_call(
        _gru_body,
        grid=(S // _T_BLK,),
        in_specs=[
            pl.BlockSpec((_T_BLK, B, 3 * H), lambda t: (t, 0, 0)),
            pl.BlockSpec((H, 3 * H), lambda t: (0, 0)),
            pl.BlockSpec((1, 3 * H), lambda t: (0, 0)),
        ],
        out_specs=pl.BlockSpec((_T_BLK, B, H), lambda t: (t, 0, 0)),
        out_shape=jax.ShapeDtypeStruct((S, B, H), jnp.bfloat16),
        scratch_shapes=[pltpu.VMEM((B, H), jnp.float32)],
        compiler_params=pltpu.CompilerParams(
            dimension_semantics=("arbitrary",)
        ),
        interpret=interpret,
    )(gi_all, w_hhT, b_hh2)


_S_BLK = 8
_N_BLK = 4096


def _fc_body(x_ref, w_ref, b_ref, o_ref):
    x = x_ref[...]
    xt = jnp.swapaxes(x, 0, 1).reshape(B * _S_BLK, H)
    y = (
        lax.dot_general(
            xt, w_ref[...], (((1,), (1,)), ((), ())),
            preferred_element_type=jnp.float32,
        )
        + b_ref[...]
    )
    o_ref[...] = y.reshape(B, _S_BLK, _N_BLK)


def _fc(outs, w_fc, b_fc2, interpret=False):
    grid = (V // _N_BLK, S // _S_BLK)
    return pl.pallas_call(
        _fc_body,
        grid=grid,
        in_specs=[
            pl.BlockSpec((_S_BLK, B, H), lambda n, s: (s, 0, 0)),
            pl.BlockSpec((_N_BLK, H), lambda n, s: (n, 0)),
            pl.BlockSpec((1, _N_BLK), lambda n, s: (0, n)),
        ],
        out_specs=pl.BlockSpec((B, _S_BLK, _N_BLK), lambda n, s: (0, s, n)),
        out_shape=jax.ShapeDtypeStruct((B, S, V), jnp.float32),
        compiler_params=pltpu.CompilerParams(
            dimension_semantics=("parallel", "arbitrary")
        ),
        interpret=interpret,
    )(outs, w_fc, b_fc2)


def kernel(embed_table, W_ih, W_hh, b_ih, b_hh, W_fc, b_fc, y_inp):
    idx = jnp.transpose(y_inp).reshape(-1).astype(jnp.int32)

    emb = _build_sc_gather(S * B)(embed_table, idx)

    bf = jnp.bfloat16
    gi = _input_proj(
        emb, jnp.transpose(W_ih).astype(bf), b_ih.reshape(1, 3 * H)
    )
    outs = _gru(
        gi.reshape(S, B, 3 * H),
        jnp.transpose(W_hh).astype(bf),
        b_hh.reshape(1, 3 * H),
    )
    logits = _fc(outs, W_fc.astype(bf), b_fc.reshape(1, V))
    h_last = outs[S - 1][None].astype(jnp.float32)
    return (logits, h_last)
